# Initial kernel scaffold; baseline (speedup 1.0000x reference)
import jax
import jax.numpy as jnp
from jax.experimental import pallas as pl


def kernel(x, weight):
    raise NotImplementedError("write your pallas kernel here")


# speedup vs baseline: 1.7911x; 1.7911x over previous
import functools

import jax
import jax.numpy as jnp
from jax.experimental import pallas as pl

_TOPK = 8
_N_EXPERTS = 64
_ROUTED_SCALE = 2.5


def _gate_body(x_ref, wt_ref, idx_ref, w_ref):
    blk = x_ref.shape[0]
    scores = jnp.dot(x_ref[...], wt_ref[...], preferred_element_type=jnp.float32)
    st = scores.T
    iota = jax.lax.broadcasted_iota(jnp.int32, (_N_EXPERTS, blk), 0)
    vals = []
    idxs = []
    for _ in range(_TOPK):
        m = jnp.max(st, axis=0, keepdims=True)
        is_max = st == m
        sel = jnp.min(jnp.where(is_max, iota, _N_EXPERTS), axis=0, keepdims=True)
        vals.append(m)
        idxs.append(sel)
        st = jnp.where(iota == sel, -jnp.inf, st)
    v = jnp.concatenate(vals, axis=0)
    ix = jnp.concatenate(idxs, axis=0)
    v = 1.0 / (1.0 + jnp.exp(-v))
    v = v * (_ROUTED_SCALE / jnp.sum(v, axis=0, keepdims=True))
    idx_ref[...] = ix
    w_ref[...] = v


@functools.partial(jax.jit, static_argnames=("blk",))
def _gate(x, weight, blk=1024):
    tokens = x.shape[0]
    dim = x.shape[1]
    wt = weight.T
    grid = (tokens // blk,)
    idx_t, w_t = pl.pallas_call(
        _gate_body,
        grid=grid,
        in_specs=[
            pl.BlockSpec((blk, dim), lambda i: (i, 0)),
            pl.BlockSpec((dim, _N_EXPERTS), lambda i: (0, 0)),
        ],
        out_specs=[
            pl.BlockSpec((_TOPK, blk), lambda i: (0, i)),
            pl.BlockSpec((_TOPK, blk), lambda i: (0, i)),
        ],
        out_shape=[
            jax.ShapeDtypeStruct((_TOPK, tokens), jnp.int32),
            jax.ShapeDtypeStruct((_TOPK, tokens), jnp.float32),
        ],
    )(x, wt)
    return idx_t.T, w_t.T


def kernel(x, weight):
    return _gate(x, weight)
